# baseline (speedup 1.0000x reference)
import jax
import jax.numpy as jnp
from jax import lax
from jax.experimental import pallas as pl
from jax.experimental.pallas import tpu as pltpu
from jax.experimental.pallas import tpu_sc as plsc

_MARGIN = 0.3
_S = 15.0
_B = 1024
_C = 100000
_RB = 64
_LOG2E = 1.4426950408889634
_LN2 = 0.6931471805599453
_NW = 32
_RPW = _B // _NW


def _sc_gather_body(costh_hbm, label_hbm, out_hbm, lab_v, buf_v, out_v, sem):
    wid = lax.axis_index("s") * 2 + lax.axis_index("c")
    base = wid * _RPW
    pltpu.sync_copy(label_hbm.at[pl.ds(base, _RPW)], lab_v)
    laba = lab_v[pl.ds(0, 16)]
    labb = lab_v[pl.ds(16, 16)]
    ta = lax.shift_right_logical(laba, 7)
    tb = lax.shift_right_logical(labb, 7)
    offa = laba & 127
    offb = labb & 127
    r16 = lax.iota(jnp.int32, 16)

    def step(k, carry):
        hita = ta == k
        hitb = tb == k

        @pl.when(jnp.any(hita) | jnp.any(hitb))
        def _():
            col0 = lax.mul(k, 128)
            pltpu.sync_copy(
                costh_hbm.at[pl.ds(base, _RPW), pl.ds(col0, 128)], buf_v)
            va = plsc.load_gather(buf_v, [r16, offa], mask=hita)
            plsc.store_scatter(out_v, [r16], va, mask=hita)
            vb = plsc.load_gather(buf_v, [r16 + 16, offb], mask=hitb)
            plsc.store_scatter(out_v, [r16 + 16], vb, mask=hitb)

        return carry

    lax.fori_loop(0, (_C + 127) // 128, step, 0)
    pltpu.sync_copy(out_v, out_hbm.at[pl.ds(base, _RPW)])


def _sc_gather(costh, label):
    mesh = plsc.VectorSubcoreMesh(core_axis_name="c", subcore_axis_name="s")
    f = pl.kernel(
        _sc_gather_body,
        out_type=jax.ShapeDtypeStruct((_B,), jnp.float32),
        mesh=mesh,
        scratch_types=[
            pltpu.VMEM((_RPW,), jnp.int32),
            pltpu.VMEM((_RPW, 128), jnp.float32),
            pltpu.VMEM((_RPW,), jnp.float32),
            pltpu.SemaphoreType.DMA,
        ],
        compiler_params=pltpu.CompilerParams(use_tc_tiling_on_sc=True,
                                             needs_layout_passes=False),
    )
    return f(costh, label)


def _tc_body(costh_ref, s_ref):
    x = costh_ref[...]
    y = x * (_S * _LOG2E)
    s = jnp.sum(jnp.exp2(y), axis=1)
    s_ref[...] = s.reshape(1, 1, _RB)


def _combine_body(s_ref, cl_ref, out_ref):
    s = s_ref[...]
    yl = cl_ref[...] * (_S * _LOG2E)
    d = _S * _MARGIN * _LOG2E
    s_corr = s - jnp.exp2(yl) + jnp.exp2(yl - d)
    total = _LN2 * jnp.sum(jnp.log2(s_corr) - (yl - d))
    out_ref[...] = total.reshape(1, 1) / _B


def kernel(costh, label):
    cl = _sc_gather(costh, label.astype(jnp.int32))
    s = pl.pallas_call(
        _tc_body,
        grid=(_B // _RB,),
        in_specs=[pl.BlockSpec((_RB, _C), lambda i: (i, 0))],
        out_specs=pl.BlockSpec((1, 1, _RB), lambda i: (i, 0, 0)),
        out_shape=jax.ShapeDtypeStruct((_B // _RB, 1, _RB), jnp.float32),
    )(costh)
    total = pl.pallas_call(
        _combine_body,
        in_specs=[
            pl.BlockSpec((_B // _RB, 1, _RB), lambda: (0, 0, 0)),
            pl.BlockSpec((_B // _RB, 1, _RB), lambda: (0, 0, 0)),
        ],
        out_specs=pl.BlockSpec((1, 1), lambda: (0, 0)),
        out_shape=jax.ShapeDtypeStruct((1, 1), jnp.float32),
    )(s, cl.reshape(_B // _RB, 1, _RB))
    return total[0, 0]


# speedup vs baseline: 1.0045x; 1.0045x over previous
import jax
import jax.numpy as jnp
from jax import lax
from jax.experimental import pallas as pl
from jax.experimental.pallas import tpu as pltpu
from jax.experimental.pallas import tpu_sc as plsc

_MARGIN = 0.3
_S = 15.0
_B = 1024
_C = 100000
_RB = 32
_LOG2E = 1.4426950408889634
_LN2 = 0.6931471805599453
_NW = 32
_RPW = _B // _NW


def _sc_gather_body(costh_hbm, label_hbm, out_hbm, lab_v, buf_v, out_v, sem):
    wid = lax.axis_index("s") * 2 + lax.axis_index("c")
    base = wid * _RPW
    pltpu.sync_copy(label_hbm.at[pl.ds(base, _RPW)], lab_v)
    laba = lab_v[pl.ds(0, 16)]
    labb = lab_v[pl.ds(16, 16)]
    ta = lax.shift_right_logical(laba, 7)
    tb = lax.shift_right_logical(labb, 7)
    offa = laba & 127
    offb = labb & 127
    r16 = lax.iota(jnp.int32, 16)

    def step(k, carry):
        hita = ta == k
        hitb = tb == k

        @pl.when(jnp.any(hita) | jnp.any(hitb))
        def _():
            col0 = lax.mul(k, 128)
            pltpu.sync_copy(
                costh_hbm.at[pl.ds(base, _RPW), pl.ds(col0, 128)], buf_v)
            va = plsc.load_gather(buf_v, [r16, offa], mask=hita)
            plsc.store_scatter(out_v, [r16], va, mask=hita)
            vb = plsc.load_gather(buf_v, [r16 + 16, offb], mask=hitb)
            plsc.store_scatter(out_v, [r16 + 16], vb, mask=hitb)

        return carry

    lax.fori_loop(0, (_C + 127) // 128, step, 0)
    pltpu.sync_copy(out_v, out_hbm.at[pl.ds(base, _RPW)])


def _sc_gather(costh, label):
    mesh = plsc.VectorSubcoreMesh(core_axis_name="c", subcore_axis_name="s")
    f = pl.kernel(
        _sc_gather_body,
        out_type=jax.ShapeDtypeStruct((_B,), jnp.float32),
        mesh=mesh,
        scratch_types=[
            pltpu.VMEM((_RPW,), jnp.int32),
            pltpu.VMEM((_RPW, 128), jnp.float32),
            pltpu.VMEM((_RPW,), jnp.float32),
            pltpu.SemaphoreType.DMA,
        ],
        compiler_params=pltpu.CompilerParams(use_tc_tiling_on_sc=True,
                                             needs_layout_passes=False),
    )
    return f(costh, label)


def _tc_body(costh_ref, s_ref):
    x = costh_ref[...]
    y = x * (_S * _LOG2E)
    s = jnp.sum(jnp.exp2(y), axis=1)
    s_ref[...] = s.reshape(1, 1, _RB)


def _combine_body(s_ref, cl_ref, out_ref):
    s = s_ref[...]
    yl = cl_ref[...] * (_S * _LOG2E)
    d = _S * _MARGIN * _LOG2E
    s_corr = s - jnp.exp2(yl) + jnp.exp2(yl - d)
    total = _LN2 * jnp.sum(jnp.log2(s_corr) - (yl - d))
    out_ref[...] = total.reshape(1, 1) / _B


def kernel(costh, label):
    cl = _sc_gather(costh, label.astype(jnp.int32))
    s = pl.pallas_call(
        _tc_body,
        grid=(_B // _RB,),
        in_specs=[pl.BlockSpec((_RB, _C), lambda i: (i, 0))],
        out_specs=pl.BlockSpec((1, 1, _RB), lambda i: (i, 0, 0)),
        out_shape=jax.ShapeDtypeStruct((_B // _RB, 1, _RB), jnp.float32),
    )(costh)
    total = pl.pallas_call(
        _combine_body,
        in_specs=[
            pl.BlockSpec((_B // _RB, 1, _RB), lambda: (0, 0, 0)),
            pl.BlockSpec((_B // _RB, 1, _RB), lambda: (0, 0, 0)),
        ],
        out_specs=pl.BlockSpec((1, 1), lambda: (0, 0)),
        out_shape=jax.ShapeDtypeStruct((1, 1), jnp.float32),
    )(s, cl.reshape(_B // _RB, 1, _RB))
    return total[0, 0]
